# Initial kernel scaffold; baseline (speedup 1.0000x reference)
import jax
import jax.numpy as jnp
from jax.experimental import pallas as pl


def kernel(x, edge_index, W1, b1, W2, b2, W3, b3):
    raise NotImplementedError("write your pallas kernel here")


# speedup vs baseline: 12.3835x; 12.3835x over previous
import functools

import jax
import jax.numpy as jnp
from jax import lax
from jax.experimental import pallas as pl
from jax.experimental.pallas import tpu as pltpu
from jax.experimental.pallas import tpu_sc as plsc

N = 10000
NP = 10240
E = 320000
K = 80
NW = 32
EPW = E // NW
CPW = EPW // K
RPT = NP // 16
ZR = 128


def _mesh():
    return plsc.VectorSubcoreMesh(
        core_axis_name="c", subcore_axis_name="s", num_cores=2, num_subcores=16
    )


def _fill(ref, rows, cols, val):
    cv = cols // 16
    v = jnp.full((16,), val, jnp.float32)

    def body(i, carry):
        r = i // cv
        cc = (i % cv) * 16
        ref[r, pl.ds(cc, 16)] = v
        return carry

    lax.fori_loop(0, rows * cv, body, 0)


@functools.lru_cache(maxsize=None)
def _make_degree():
    @functools.partial(
        pl.kernel,
        out_type=jax.ShapeDtypeStruct((2, NP, 16), jnp.float32),
        mesh=_mesh(),
        scratch_types=[
            pltpu.VMEM((K,), jnp.int32),
            pltpu.VMEM((K, 16), jnp.float32),
            pltpu.VMEM((RPT, 16), jnp.float32),
            pltpu.VMEM_SHARED((NP, 16), jnp.float32),
        ],
        compiler_params=pltpu.CompilerParams(use_tc_tiling_on_sc=False),
    )
    def degree(dst1d, out, idx_v, ones_v, z_v, acc):
        c = lax.axis_index("c")
        s = lax.axis_index("s")
        wid = s * 2 + c
        _fill(z_v, RPT, 16, 0.0)
        _fill(ones_v, K, 16, 1.0)
        roff = pl.multiple_of(s * RPT, 8)
        pltpu.sync_copy(z_v, acc.at[pl.ds(roff, RPT)])
        plsc.subcore_barrier()

        def body(i, carry):
            base = pl.multiple_of(wid * EPW + i * K, 8)
            pltpu.sync_copy(dst1d.at[pl.ds(base, K)], idx_v)
            pltpu.sync_copy(ones_v, acc.at[idx_v], add=True)
            return carry

        lax.fori_loop(0, CPW, body, 0)
        plsc.subcore_barrier()
        pltpu.sync_copy(acc.at[pl.ds(roff, RPT)], out.at[c, pl.ds(roff, RPT)])

    return degree


@functools.lru_cache(maxsize=None)
def _make_prop(D):
    @functools.partial(
        pl.kernel,
        out_type=jax.ShapeDtypeStruct((2, NP, D), jnp.float32),
        mesh=_mesh(),
        scratch_types=[
            pltpu.VMEM((K,), jnp.int32),
            pltpu.VMEM((K,), jnp.int32),
            pltpu.VMEM((K, D), jnp.float32),
            pltpu.VMEM((ZR, D), jnp.float32),
            pltpu.VMEM_SHARED((NP, D), jnp.float32),
            pltpu.SemaphoreType.DMA,
        ],
        compiler_params=pltpu.CompilerParams(use_tc_tiling_on_sc=False),
    )
    def prop(hp, src1d, dst1d, out, si_v, di_v, rows_v, z_v, acc, sem):
        c = lax.axis_index("c")
        s = lax.axis_index("s")
        wid = s * 2 + c
        _fill(z_v, ZR, D, 0.0)
        roff = pl.multiple_of(s * RPT, 8)
        for r in range(RPT // ZR):
            pltpu.sync_copy(z_v, acc.at[pl.ds(roff + r * ZR, ZR)])
        plsc.subcore_barrier()

        def body(i, carry):
            base = pl.multiple_of(wid * EPW + i * K, 8)
            pltpu.sync_copy(src1d.at[pl.ds(base, K)], si_v)
            pltpu.sync_copy(dst1d.at[pl.ds(base, K)], di_v)
            pltpu.async_copy(hp.at[si_v], rows_v, sem).wait()
            pltpu.sync_copy(rows_v, acc.at[di_v], add=True)
            return carry

        lax.fori_loop(0, CPW, body, 0)
        plsc.subcore_barrier()
        pltpu.sync_copy(acc.at[pl.ds(roff, RPT)], out.at[c, pl.ds(roff, RPT)])

    return prop


_B = 1000


def _tc_first_body(x_ref, w_ref, degp_ref, hp_ref, dinv_ref):
    deg = degp_ref[0, :, 0:1] + degp_ref[1, :, 0:1] + 1.0
    dinv = lax.rsqrt(deg)
    t = jnp.dot(x_ref[...], w_ref[...], preferred_element_type=jnp.float32)
    hp_ref[...] = t * dinv
    dinv_ref[...] = dinv


def _tc_first(x, W1, degp):
    Din, Dout = W1.shape
    return pl.pallas_call(
        _tc_first_body,
        grid=(N // _B,),
        in_specs=[
            pl.BlockSpec((_B, Din), lambda i: (i, 0)),
            pl.BlockSpec((Din, Dout), lambda i: (0, 0)),
            pl.BlockSpec((2, _B, 16), lambda i: (0, i, 0)),
        ],
        out_specs=[
            pl.BlockSpec((_B, Dout), lambda i: (i, 0)),
            pl.BlockSpec((_B, 1), lambda i: (i, 0)),
        ],
        out_shape=[
            jax.ShapeDtypeStruct((N, Dout), jnp.float32),
            jax.ShapeDtypeStruct((N, 1), jnp.float32),
        ],
    )(x, W1, degp)


def _tc_mid_body(p_ref, hp_ref, dinv_ref, b_ref, w_ref, hpn_ref):
    dinv = dinv_ref[...]
    sm = p_ref[0] + p_ref[1] + hp_ref[...]
    a = jnp.maximum(sm * dinv + b_ref[...], 0.0)
    t = jnp.dot(a, w_ref[...], preferred_element_type=jnp.float32)
    hpn_ref[...] = t * dinv


def _tc_mid(p, hp, dinv, b, W):
    Din, Dout = W.shape
    return pl.pallas_call(
        _tc_mid_body,
        grid=(N // _B,),
        in_specs=[
            pl.BlockSpec((2, _B, Din), lambda i: (0, i, 0)),
            pl.BlockSpec((_B, Din), lambda i: (i, 0)),
            pl.BlockSpec((_B, 1), lambda i: (i, 0)),
            pl.BlockSpec((1, Din), lambda i: (0, 0)),
            pl.BlockSpec((Din, Dout), lambda i: (0, 0)),
        ],
        out_specs=pl.BlockSpec((_B, Dout), lambda i: (i, 0)),
        out_shape=jax.ShapeDtypeStruct((N, Dout), jnp.float32),
    )(p, hp, dinv, b, W)


def _tc_last_body(p_ref, hp_ref, dinv_ref, b_ref, out_ref):
    dinv = dinv_ref[...]
    sm = p_ref[0] + p_ref[1] + hp_ref[...]
    out_ref[...] = sm * dinv + b_ref[...]


def _tc_last(p, hp, dinv, b):
    D = hp.shape[1]
    return pl.pallas_call(
        _tc_last_body,
        grid=(N // _B,),
        in_specs=[
            pl.BlockSpec((2, _B, D), lambda i: (0, i, 0)),
            pl.BlockSpec((_B, D), lambda i: (i, 0)),
            pl.BlockSpec((_B, 1), lambda i: (i, 0)),
            pl.BlockSpec((1, D), lambda i: (0, 0)),
        ],
        out_specs=pl.BlockSpec((_B, D), lambda i: (i, 0)),
        out_shape=jax.ShapeDtypeStruct((N, D), jnp.float32),
    )(p, hp, dinv, b)


def kernel(x, edge_index, W1, b1, W2, b2, W3, b3):
    ei = edge_index.astype(jnp.int32)
    src1d = ei[0]
    dst1d = ei[1]

    degp = _make_degree()(dst1d)
    hp1, dinv = _tc_first(x, W1, degp)
    p1 = _make_prop(W1.shape[1])(hp1, src1d, dst1d)
    hp2 = _tc_mid(p1, hp1, dinv, b1.reshape(1, -1), W2)
    p2 = _make_prop(W2.shape[1])(hp2, src1d, dst1d)
    hp3 = _tc_mid(p2, hp2, dinv, b2.reshape(1, -1), W3)
    p3 = _make_prop(W3.shape[1])(hp3, src1d, dst1d)
    return _tc_last(p3, hp3, dinv, b3.reshape(1, -1))


# speedup vs baseline: 22.5905x; 1.8242x over previous
import functools

import jax
import jax.numpy as jnp
from jax import lax
from jax.experimental import pallas as pl
from jax.experimental.pallas import tpu as pltpu
from jax.experimental.pallas import tpu_sc as plsc

N = 10000
NP = 10240
E = 320000
K = 100
NW = 32
EPW = E // NW
CPW = EPW // K
RPT = NP // 16
ZR = 64

K_BY_D = {128: 50, 64: 100, 32: 100}


def _mesh():
    return plsc.VectorSubcoreMesh(
        core_axis_name="c", subcore_axis_name="s", num_cores=2, num_subcores=16
    )


def _fill(ref, rows, cols, val):
    cv = cols // 16
    v = jnp.full((16,), val, jnp.float32)

    def body(i, carry):
        r = i // cv
        cc = (i % cv) * 16
        ref[r, pl.ds(cc, 16)] = v
        return carry

    lax.fori_loop(0, rows * cv, body, 0)


@functools.lru_cache(maxsize=None)
def _make_degree():
    @functools.partial(
        pl.kernel,
        out_type=jax.ShapeDtypeStruct((2, NP, 16), jnp.float32),
        mesh=_mesh(),
        scratch_types=[
            pltpu.VMEM((CPW, K), jnp.int32),
            pltpu.VMEM((K, 16), jnp.float32),
            pltpu.VMEM((RPT, 16), jnp.float32),
            pltpu.VMEM_SHARED((NP, 16), jnp.float32),
            pltpu.SemaphoreType.DMA,
        ],
        compiler_params=pltpu.CompilerParams(use_tc_tiling_on_sc=False),
    )
    def degree(dst3d, out, di_all, ones_v, z_v, acc, sem):
        c = lax.axis_index("c")
        s = lax.axis_index("s")
        wid = s * 2 + c
        _fill(z_v, RPT, 16, 0.0)
        _fill(ones_v, K, 16, 1.0)
        roff = pl.multiple_of(s * RPT, 8)
        pltpu.sync_copy(z_v, acc.at[pl.ds(roff, RPT)])
        pltpu.sync_copy(dst3d.at[wid], di_all)
        plsc.subcore_barrier()

        def grp(g, carry):
            cps = [
                pltpu.async_copy(
                    ones_v, acc.at[di_all.at[g * 4 + b]], sem, add=True
                )
                for b in range(4)
            ]
            for cp in cps:
                cp.wait()
            return carry

        lax.fori_loop(0, CPW // 4, grp, 0)
        plsc.subcore_barrier()
        pltpu.sync_copy(acc.at[pl.ds(roff, RPT)], out.at[c, pl.ds(roff, RPT)])

    return degree


@functools.lru_cache(maxsize=None)
def _make_prop(D):
    kd = K_BY_D[D]
    cpw = EPW // kd
    npair = cpw // 2

    @functools.partial(
        pl.kernel,
        out_type=jax.ShapeDtypeStruct((2, NP, D), jnp.float32),
        mesh=_mesh(),
        scratch_types=[
            pltpu.VMEM((cpw, kd), jnp.int32),
            pltpu.VMEM((cpw, kd), jnp.int32),
            pltpu.VMEM((kd, D), jnp.float32),
            pltpu.VMEM((kd, D), jnp.float32),
            pltpu.VMEM((ZR, D), jnp.float32),
            pltpu.VMEM_SHARED((NP, D), jnp.float32),
            pltpu.SemaphoreType.DMA,
            pltpu.SemaphoreType.DMA,
        ],
        compiler_params=pltpu.CompilerParams(use_tc_tiling_on_sc=False),
    )
    def prop(hp, src3d, dst3d, out, si_all, di_all, rows0, rows1, z_v, acc,
             sem0, sem1):
        c = lax.axis_index("c")
        s = lax.axis_index("s")
        wid = s * 2 + c
        _fill(z_v, ZR, D, 0.0)
        roff = pl.multiple_of(s * RPT, 8)
        for r in range(RPT // ZR):
            pltpu.sync_copy(z_v, acc.at[pl.ds(roff + r * ZR, ZR)])
        pltpu.sync_copy(src3d.at[wid], si_all)
        pltpu.sync_copy(dst3d.at[wid], di_all)
        plsc.subcore_barrier()

        pltpu.async_copy(hp.at[si_all.at[0]], rows0, sem0)

        def pair(g, carry):
            i0 = g * 2
            pltpu.make_async_copy(hp.at[si_all.at[i0]], rows0, sem0).wait()
            pltpu.async_copy(hp.at[si_all.at[i0 + 1]], rows1, sem1)
            pltpu.sync_copy(rows0, acc.at[di_all.at[i0]], add=True)
            pltpu.make_async_copy(
                hp.at[si_all.at[i0 + 1]], rows1, sem1
            ).wait()

            @pl.when(g + 1 < npair)
            def _():
                pltpu.async_copy(hp.at[si_all.at[i0 + 2]], rows0, sem0)

            pltpu.sync_copy(rows1, acc.at[di_all.at[i0 + 1]], add=True)
            return carry

        lax.fori_loop(0, npair, pair, 0)
        plsc.subcore_barrier()
        pltpu.sync_copy(acc.at[pl.ds(roff, RPT)], out.at[c, pl.ds(roff, RPT)])

    return prop


_B = 1000


def _tc_first_body(x_ref, w_ref, degp_ref, hp_ref, dinv_ref):
    deg = degp_ref[0, :, 0:1] + degp_ref[1, :, 0:1] + 1.0
    dinv = lax.rsqrt(deg)
    t = jnp.dot(x_ref[...], w_ref[...], preferred_element_type=jnp.float32)
    hp_ref[...] = t * dinv
    dinv_ref[...] = dinv


def _tc_first(x, W1, degp):
    Din, Dout = W1.shape
    return pl.pallas_call(
        _tc_first_body,
        grid=(N // _B,),
        in_specs=[
            pl.BlockSpec((_B, Din), lambda i: (i, 0)),
            pl.BlockSpec((Din, Dout), lambda i: (0, 0)),
            pl.BlockSpec((2, _B, 16), lambda i: (0, i, 0)),
        ],
        out_specs=[
            pl.BlockSpec((_B, Dout), lambda i: (i, 0)),
            pl.BlockSpec((_B, 1), lambda i: (i, 0)),
        ],
        out_shape=[
            jax.ShapeDtypeStruct((N, Dout), jnp.float32),
            jax.ShapeDtypeStruct((N, 1), jnp.float32),
        ],
    )(x, W1, degp)


def _tc_mid_body(p_ref, hp_ref, dinv_ref, b_ref, w_ref, hpn_ref):
    dinv = dinv_ref[...]
    sm = p_ref[0] + p_ref[1] + hp_ref[...]
    a = jnp.maximum(sm * dinv + b_ref[...], 0.0)
    t = jnp.dot(a, w_ref[...], preferred_element_type=jnp.float32)
    hpn_ref[...] = t * dinv


def _tc_mid(p, hp, dinv, b, W):
    Din, Dout = W.shape
    return pl.pallas_call(
        _tc_mid_body,
        grid=(N // _B,),
        in_specs=[
            pl.BlockSpec((2, _B, Din), lambda i: (0, i, 0)),
            pl.BlockSpec((_B, Din), lambda i: (i, 0)),
            pl.BlockSpec((_B, 1), lambda i: (i, 0)),
            pl.BlockSpec((1, Din), lambda i: (0, 0)),
            pl.BlockSpec((Din, Dout), lambda i: (0, 0)),
        ],
        out_specs=pl.BlockSpec((_B, Dout), lambda i: (i, 0)),
        out_shape=jax.ShapeDtypeStruct((N, Dout), jnp.float32),
    )(p, hp, dinv, b, W)


def _tc_last_body(p_ref, hp_ref, dinv_ref, b_ref, out_ref):
    dinv = dinv_ref[...]
    sm = p_ref[0] + p_ref[1] + hp_ref[...]
    out_ref[...] = sm * dinv + b_ref[...]


def _tc_last(p, hp, dinv, b):
    D = hp.shape[1]
    return pl.pallas_call(
        _tc_last_body,
        grid=(N // _B,),
        in_specs=[
            pl.BlockSpec((2, _B, D), lambda i: (0, i, 0)),
            pl.BlockSpec((_B, D), lambda i: (i, 0)),
            pl.BlockSpec((_B, 1), lambda i: (i, 0)),
            pl.BlockSpec((1, D), lambda i: (0, 0)),
        ],
        out_specs=pl.BlockSpec((_B, D), lambda i: (i, 0)),
        out_shape=jax.ShapeDtypeStruct((N, D), jnp.float32),
    )(p, hp, dinv, b)


def _edges3d(v, D):
    kd = K_BY_D[D]
    return v.reshape(NW, EPW // kd, kd)


def kernel(x, edge_index, W1, b1, W2, b2, W3, b3):
    ei = edge_index.astype(jnp.int32)
    src, dst = ei[0], ei[1]

    degp = _make_degree()(dst.reshape(NW, CPW, K))
    hp1, dinv = _tc_first(x, W1, degp)
    d1 = W1.shape[1]
    p1 = _make_prop(d1)(hp1, _edges3d(src, d1), _edges3d(dst, d1))
    hp2 = _tc_mid(p1, hp1, dinv, b1.reshape(1, -1), W2)
    d2 = W2.shape[1]
    p2 = _make_prop(d2)(hp2, _edges3d(src, d2), _edges3d(dst, d2))
    hp3 = _tc_mid(p2, hp2, dinv, b2.reshape(1, -1), W3)
    d3 = W3.shape[1]
    p3 = _make_prop(d3)(hp3, _edges3d(src, d3), _edges3d(dst, d3))
    return _tc_last(p3, hp3, dinv, b3.reshape(1, -1))
